# baseline (speedup 1.0000x reference)
import functools

import jax
import jax.numpy as jnp
from jax import lax
from jax.experimental import pallas as pl
from jax.experimental.pallas import tpu as pltpu
from jax.experimental.pallas import tpu_sc as plsc

VOCAB = 100000
EMBED = 16
BATCH = 1024
CTX = 50

NC = 2
NS = 16
NW = NC * NS
B_PER_W = BATCH // NW
ITEMS_PER_CHUNK = 2
CHUNK_I = ITEMS_PER_CHUNK * CTX
CHUNKS = B_PER_W // ITEMS_PER_CHUNK


def _tree_sum(vals):
  while len(vals) > 1:
    nxt = [vals[i] + vals[i + 1] for i in range(0, len(vals) - 1, 2)]
    if len(vals) % 2:
      nxt.append(vals[-1])
    vals = nxt
  return vals[0]


@functools.cache
def _make_sc_gather_mean():
  @functools.partial(
      pl.kernel,
      out_type=jax.ShapeDtypeStruct((BATCH, EMBED), jnp.float32),
      mesh=plsc.VectorSubcoreMesh(core_axis_name="c", subcore_axis_name="s",
                                  num_cores=NC, num_subcores=NS),
      scratch_types=[
          pltpu.VMEM((CHUNKS, CHUNK_I), jnp.int32),
          pltpu.VMEM((CHUNK_I, EMBED), jnp.float32),
          pltpu.VMEM((B_PER_W, EMBED), jnp.float32),
          pltpu.SemaphoreType.DMA,
      ],
      compiler_params=pltpu.CompilerParams(use_tc_tiling_on_sc=False),
  )
  def _sc_gather_mean(ctx_hbm, table_hbm, ebar_hbm, idx_v, rows_v, acc_v, sem):
    wid = lax.axis_index("s") * NC + lax.axis_index("c")
    pltpu.sync_copy(ctx_hbm.at[wid], idx_v)
    for k in range(CHUNKS):
      pltpu.async_copy(table_hbm.at[idx_v.at[k]], rows_v, sem).wait()
      for t in range(ITEMS_PER_CHUNK):
        acc = _tree_sum([rows_v[t * CTX + j, :] for j in range(CTX)])
        acc_v[k * ITEMS_PER_CHUNK + t, :] = acc * (1.0 / CTX)
    pltpu.sync_copy(acc_v, ebar_hbm.at[pl.ds(wid * B_PER_W, B_PER_W)])

  return _sc_gather_mean


V_BLK = 1024
V_GRID = (VOCAB + V_BLK - 1) // V_BLK


def _mm_body(e_ref, u_ref, o_ref):
  o_ref[...] = jnp.dot(e_ref[...], u_ref[...],
                       preferred_element_type=jnp.float32)


_tc_matmul = pl.pallas_call(
    _mm_body,
    grid=(V_GRID,),
    in_specs=[
        pl.BlockSpec((BATCH, EMBED), lambda i: (0, 0)),
        pl.BlockSpec((EMBED, V_BLK), lambda i: (0, i)),
    ],
    out_specs=pl.BlockSpec((BATCH, V_BLK), lambda i: (0, i)),
    out_shape=jax.ShapeDtypeStruct((BATCH, VOCAB), jnp.float32),
    compiler_params=pltpu.CompilerParams(
        dimension_semantics=("arbitrary",)),
)


def kernel(context, embeddings, U_T):
  ctx = context.astype(jnp.int32).reshape(NW, CHUNKS, CHUNK_I)
  e_bar = _make_sc_gather_mean()(ctx, embeddings)
  return _tc_matmul(e_bar, U_T.T)


# speedup vs baseline: 1.0441x; 1.0441x over previous
import functools

import jax
import jax.numpy as jnp
from jax import lax
from jax.experimental import pallas as pl
from jax.experimental.pallas import tpu as pltpu
from jax.experimental.pallas import tpu_sc as plsc

VOCAB = 100000
EMBED = 16
BATCH = 1024
CTX = 50

NC = 2
NS = 16
NW = NC * NS
B_PER_W = BATCH // NW
ITEMS_PER_CHUNK = 2
CHUNK_I = ITEMS_PER_CHUNK * CTX
CHUNKS = B_PER_W // ITEMS_PER_CHUNK


def _tree_sum(vals):
  while len(vals) > 1:
    nxt = [vals[i] + vals[i + 1] for i in range(0, len(vals) - 1, 2)]
    if len(vals) % 2:
      nxt.append(vals[-1])
    vals = nxt
  return vals[0]


@functools.cache
def _make_sc_gather_mean():
  @functools.partial(
      pl.kernel,
      out_type=jax.ShapeDtypeStruct((BATCH, EMBED), jnp.float32),
      mesh=plsc.VectorSubcoreMesh(core_axis_name="c", subcore_axis_name="s",
                                  num_cores=NC, num_subcores=NS),
      scratch_types=[
          pltpu.VMEM((CHUNKS, CHUNK_I), jnp.int32),
          pltpu.VMEM((CHUNK_I, EMBED), jnp.float32),
          pltpu.VMEM((B_PER_W, EMBED), jnp.float32),
          pltpu.SemaphoreType.DMA,
      ],
      compiler_params=pltpu.CompilerParams(use_tc_tiling_on_sc=False),
  )
  def _sc_gather_mean(ctx_hbm, table_hbm, ebar_hbm, idx_v, rows_v, acc_v, sem):
    wid = lax.axis_index("s") * NC + lax.axis_index("c")
    pltpu.sync_copy(ctx_hbm.at[wid], idx_v)
    for k in range(CHUNKS):
      pltpu.async_copy(table_hbm.at[idx_v.at[k]], rows_v, sem).wait()
      for t in range(ITEMS_PER_CHUNK):
        acc = _tree_sum([rows_v[t * CTX + j, :] for j in range(CTX)])
        acc_v[k * ITEMS_PER_CHUNK + t, :] = acc * (1.0 / CTX)
    pltpu.sync_copy(acc_v, ebar_hbm.at[pl.ds(wid * B_PER_W, B_PER_W)])

  return _sc_gather_mean


V_BLK = 4096
V_GRID = (VOCAB + V_BLK - 1) // V_BLK


def _mm_body(e_ref, u_ref, o_ref):
  o_ref[...] = jnp.dot(e_ref[...], u_ref[...],
                       preferred_element_type=jnp.float32)


_tc_matmul = pl.pallas_call(
    _mm_body,
    grid=(V_GRID,),
    in_specs=[
        pl.BlockSpec((BATCH, EMBED), lambda i: (0, 0)),
        pl.BlockSpec((EMBED, V_BLK), lambda i: (0, i)),
    ],
    out_specs=pl.BlockSpec((BATCH, V_BLK), lambda i: (0, i)),
    out_shape=jax.ShapeDtypeStruct((BATCH, VOCAB), jnp.float32),
    compiler_params=pltpu.CompilerParams(
        dimension_semantics=("parallel",)),
)


def kernel(context, embeddings, U_T):
  ctx = context.astype(jnp.int32).reshape(NW, CHUNKS, CHUNK_I)
  e_bar = _make_sc_gather_mean()(ctx, embeddings)
  return _tc_matmul(e_bar, U_T.T)


# speedup vs baseline: 2.4775x; 2.3729x over previous
import functools

import jax
import jax.numpy as jnp
from jax import lax
from jax.experimental import pallas as pl
from jax.experimental.pallas import tpu as pltpu
from jax.experimental.pallas import tpu_sc as plsc

VOCAB = 100000
EMBED = 16
BATCH = 1024
CTX = 50

NC = 2
NS = 16
NW = NC * NS
B_PER_W = BATCH // NW
ITEMS_PER_CHUNK = 2
CHUNK_I = ITEMS_PER_CHUNK * CTX
CHUNKS = B_PER_W // ITEMS_PER_CHUNK


def _tree_sum(vals):
  while len(vals) > 1:
    nxt = [vals[i] + vals[i + 1] for i in range(0, len(vals) - 1, 2)]
    if len(vals) % 2:
      nxt.append(vals[-1])
    vals = nxt
  return vals[0]


@functools.cache
def _make_sc_gather_mean():
  @functools.partial(
      pl.kernel,
      out_type=jax.ShapeDtypeStruct((BATCH, EMBED), jnp.float32),
      mesh=plsc.VectorSubcoreMesh(core_axis_name="c", subcore_axis_name="s",
                                  num_cores=NC, num_subcores=NS),
      scratch_types=[
          pltpu.VMEM((CHUNKS, CHUNK_I), jnp.int32),
          pltpu.VMEM((CHUNK_I, EMBED), jnp.float32),
          pltpu.VMEM((B_PER_W, EMBED), jnp.float32),
          pltpu.SemaphoreType.DMA,
      ],
      compiler_params=pltpu.CompilerParams(use_tc_tiling_on_sc=False),
  )
  def _sc_gather_mean(ctx_hbm, table_hbm, ebar_hbm, idx_v, rows_v, acc_v, sem):
    wid = lax.axis_index("s") * NC + lax.axis_index("c")
    pltpu.sync_copy(ctx_hbm.at[wid], idx_v)
    for k in range(CHUNKS):
      pltpu.async_copy(table_hbm.at[idx_v.at[k]], rows_v, sem).wait()
      for t in range(ITEMS_PER_CHUNK):
        acc = _tree_sum([rows_v[t * CTX + j, :] for j in range(CTX)])
        acc_v[k * ITEMS_PER_CHUNK + t, :] = acc * (1.0 / CTX)
    pltpu.sync_copy(acc_v, ebar_hbm.at[pl.ds(wid * B_PER_W, B_PER_W)])

  return _sc_gather_mean


V_BLK = 2000
V_GRID = VOCAB // V_BLK


def _mm_body(u_ref, e_ref, o_ref):
  o_ref[...] = jnp.dot(u_ref[...], e_ref[...],
                       preferred_element_type=jnp.float32)


_tc_matmul_t = pl.pallas_call(
    _mm_body,
    grid=(V_GRID,),
    in_specs=[
        pl.BlockSpec((V_BLK, EMBED), lambda i: (i, 0)),
        pl.BlockSpec((EMBED, BATCH), lambda i: (0, 0)),
    ],
    out_specs=pl.BlockSpec((V_BLK, BATCH), lambda i: (i, 0)),
    out_shape=jax.ShapeDtypeStruct((VOCAB, BATCH), jnp.float32),
    compiler_params=pltpu.CompilerParams(
        dimension_semantics=("parallel",)),
)


def kernel(context, embeddings, U_T):
  ctx = context.astype(jnp.int32).reshape(NW, CHUNKS, CHUNK_I)
  e_bar = _make_sc_gather_mean()(ctx, embeddings)
  out_t = _tc_matmul_t(U_T, e_bar.T)
  return out_t.T
